# Initial kernel scaffold; baseline (speedup 1.0000x reference)
import jax
import jax.numpy as jnp
from jax.experimental import pallas as pl


def kernel(x_, edge_index, edge_attr, mask, W_t0, b_t0, W_t1, b_t1, W_gat, att_src, att_dst, b_gat, gn_gamma, gn_beta, gn_alpha, W_c0, b_c0, W_c1, b_c1):
    raise NotImplementedError("write your pallas kernel here")


# speedup vs baseline: 9.9514x; 9.9514x over previous
import dataclasses
import functools

import jax
import jax.numpy as jnp
from jax import lax
from jax.experimental import pallas as pl
from jax.experimental.pallas import tpu as pltpu
from jax.experimental.pallas import tpu_sc as plsc

N = 10000
E = 320000
D = 128
H = 64
Z = 0.8
L = 16
NS = 16
NC = 2
C = 128
NCHUNK = 160
PT = NCHUNK * C
EP = NS * PT
NPAD = 10240
NB = 8


def _dense_a_body(x_ref, mf_ref, wt0_ref, bt0_ref, wt1_ref, bt1_ref, wg_ref,
                  asv_ref, adv_ref, h_ref, asrc_ref, adst_ref):
    x_ = x_ref[...]
    x1 = jnp.maximum(jnp.dot(x_, wt1_ref[...],
                             preferred_element_type=jnp.float32)
                     + bt1_ref[...], 0.0)
    x0 = jnp.maximum(jnp.dot(x_, wt0_ref[...],
                             preferred_element_type=jnp.float32)
                     + bt0_ref[...], 0.0)
    mf = mf_ref[...]
    x = mf * (Z * x1 + (1 - Z) * x0) + (1 - mf) * (Z * x0 + (1 - Z) * x1)
    h = jnp.dot(x, wg_ref[...], preferred_element_type=jnp.float32)
    h_ref[0:N, :] = h[:, :H]
    h_ref[N:2 * N, :] = h[:, H:]
    asrc_ref[...] = jnp.sum(h * asv_ref[...], axis=1, keepdims=True)
    adst_ref[...] = jnp.sum(h * adv_ref[...], axis=1, keepdims=True)


def _dense_a(x_, mf, W_t0, b_t0, W_t1, b_t1, W_gat, att_src, att_dst):
    return pl.pallas_call(
        _dense_a_body,
        out_shape=(
            jax.ShapeDtypeStruct((2 * N, H), jnp.float32),
            jax.ShapeDtypeStruct((N, 1), jnp.float32),
            jax.ShapeDtypeStruct((N, 1), jnp.float32),
        ),
    )(x_, mf, W_t0, b_t0.reshape(1, D), W_t1, b_t1.reshape(1, D), W_gat,
      att_src.reshape(1, D), att_dst.reshape(1, D))


def _sc_edge_body(h_hbm, asrc_hbm, adst_hbm, src_hbm, dst_hbm, attr_hbm,
                  out_hbm,
                  asrc_t, adst_t, e_t, rows_t, src_b, dst_b, attr_b,
                  out_sh, den_sh, max_sh, sem):
    c = lax.axis_index("c")
    s = lax.axis_index("s")
    row0 = s * NCHUNK

    pltpu.sync_copy(asrc_hbm, asrc_t)
    pltpu.sync_copy(adst_hbm, adst_t)

    zz = jnp.zeros((L,), jnp.float32)

    @pl.loop(0, C)
    def _(i):
        for q in range(H // L):
            rows_t[i, pl.ds(q * L, L)] = zz

    for q in range(5):
        pltpu.sync_copy(rows_t, out_sh.at[pl.ds(s * 640 + q * C, C)])
    for q in range(10):
        pltpu.sync_copy(rows_t.at[0], den_sh.at[pl.ds(s * 640 + q * H, H)])

    def p1_blk(b, mx):
        pltpu.sync_copy(src_hbm.at[pl.ds(row0 + b * NB, NB)], src_b)
        pltpu.sync_copy(dst_hbm.at[pl.ds(row0 + b * NB, NB)], dst_b)

        def p1_row(jj, mx):
            def p1_step(k, mx):
                sl = pl.ds(k * L, L)
                s16 = src_b[jj, sl]
                d16 = dst_b[jj, sl]
                t = (plsc.load_gather(asrc_t, [s16])
                     + plsc.load_gather(adst_t, [d16]))
                e16 = jnp.where(t >= 0.0, t, 0.2 * t)
                e_t[b * NB + jj, sl] = e16
                return jnp.maximum(mx, e16)
            return lax.fori_loop(0, C // L, p1_step, mx)
        return lax.fori_loop(0, NB, p1_row, mx)

    mx = lax.fori_loop(0, NCHUNK // NB, p1_blk,
                       jnp.full((L,), -1e30, jnp.float32))

    adst_t[pl.ds(0, L)] = mx
    pltpu.sync_copy(adst_t.at[pl.ds(0, L)], max_sh.at[pl.ds(s * L, L)])
    plsc.subcore_barrier()
    pltpu.sync_copy(max_sh, adst_t.at[pl.ds(0, NS * L)])

    def mx_step(i, mv):
        return jnp.maximum(mv, adst_t[pl.ds(i * L, L)])

    M = jnp.max(lax.fori_loop(0, NS, mx_step,
                              jnp.full((L,), -1e30, jnp.float32)))

    @pl.loop(0, NCHUNK // NB)
    def _(b):
        pltpu.sync_copy(dst_hbm.at[pl.ds(row0 + b * NB, NB)], dst_b)
        pltpu.sync_copy(attr_hbm.at[pl.ds(row0 + b * NB, NB)], attr_b)

        @pl.loop(0, NB)
        def _(jj):
            j = b * NB + jj

            @pl.loop(0, C // L)
            def _(k):
                sl = pl.ds(k * L, L)
                e_t[j, sl] = jnp.exp(e_t[j, sl] - M) * attr_b[jj, sl]

            pltpu.sync_copy(e_t.at[j], den_sh.at[dst_b.at[jj]], add=True)

    plsc.subcore_barrier()
    pltpu.sync_copy(den_sh.at[pl.ds(0, N)], asrc_t)

    @pl.loop(0, NCHUNK // NB)
    def _(b):
        pltpu.sync_copy(src_hbm.at[pl.ds(row0 + b * NB, NB)], src_b)
        pltpu.sync_copy(dst_hbm.at[pl.ds(row0 + b * NB, NB)], dst_b)

        @pl.loop(0, NB)
        def _(jj):
            j = b * NB + jj

            @pl.loop(0, C // L)
            def _(k):
                sl = pl.ds(k * L, L)
                den = plsc.load_gather(asrc_t, [dst_b[jj, sl]])
                e_t[j, sl] = e_t[j, sl] / (den + 1e-16)
                src_b[jj, sl] = src_b[jj, sl] + c * N

            pltpu.sync_copy(h_hbm.at[src_b.at[jj]], rows_t)

            @pl.loop(0, C // L)
            def _(k):
                base = k * L
                for i in range(L):
                    asp = plsc.load_gather(
                        e_t, [jnp.full((L,), j, jnp.int32),
                              jnp.full((L,), base + i, jnp.int32)])
                    for q in range(H // L):
                        sl = pl.ds(q * L, L)
                        rows_t[base + i, sl] = rows_t[base + i, sl] * asp

            pltpu.sync_copy(rows_t, out_sh.at[dst_b.at[jj]], add=True)

    plsc.subcore_barrier()

    for q in range(5):
        r0 = s * 640 + q * C
        pltpu.sync_copy(out_sh.at[pl.ds(r0, C)],
                        out_hbm.at[pl.ds(c * NPAD + r0, C)])


def _sc_edge(h2, a_src, a_dst, srcp, dstp, attrp):
    mesh = plsc.VectorSubcoreMesh(core_axis_name="c", subcore_axis_name="s")
    cp = pltpu.CompilerParams()
    if "needs_layout_passes" in pltpu.CompilerParams.__dataclass_fields__:
        cp = dataclasses.replace(cp, needs_layout_passes=False)
    if "use_tc_tiling_on_sc" in pltpu.CompilerParams.__dataclass_fields__:
        cp = dataclasses.replace(cp, use_tc_tiling_on_sc=False)
    kern = functools.partial(
        pl.kernel,
        compiler_params=cp,
        out_type=jax.ShapeDtypeStruct((2 * NPAD, H), jnp.float32),
        mesh=mesh,
        scratch_types=[
            pltpu.VMEM((N,), jnp.float32),
            pltpu.VMEM((N,), jnp.float32),
            pltpu.VMEM((NCHUNK, C), jnp.float32),
            pltpu.VMEM((C, H), jnp.float32),
            pltpu.VMEM((NB, C), jnp.int32),
            pltpu.VMEM((NB, C), jnp.int32),
            pltpu.VMEM((NB, C), jnp.float32),
            pltpu.VMEM_SHARED((NPAD, H), jnp.float32),
            pltpu.VMEM_SHARED((NPAD,), jnp.float32),
            pltpu.VMEM_SHARED((NS * L,), jnp.float32),
            pltpu.SemaphoreType.DMA,
        ],
    )(_sc_edge_body)
    return kern(h2, a_src, a_dst, srcp, dstp, attrp)


def _dense_c_body(o2_ref, x_ref, mf_ref, bg_ref, gg_ref, gb_ref, ga_ref,
                  wc0a_ref, wc0b_ref, bc0_ref, wc1a_ref, wc1b_ref, bc1_ref,
                  o_ref):
    out = jnp.concatenate([o2_ref[0], o2_ref[1]], axis=-1) + bg_ref[...]
    mean = jnp.mean(out, axis=0, keepdims=True)
    centered = out - ga_ref[...] * mean
    var = jnp.mean(centered * centered, axis=0, keepdims=True)
    outn = gg_ref[...] * centered * jax.lax.rsqrt(var + 1e-5) + gb_ref[...]
    x_ = x_ref[...]
    c1 = (jnp.dot(outn, wc1a_ref[...], preferred_element_type=jnp.float32)
          + jnp.dot(x_, wc1b_ref[...], preferred_element_type=jnp.float32)
          + bc1_ref[...])
    c0 = (jnp.dot(outn, wc0a_ref[...], preferred_element_type=jnp.float32)
          + jnp.dot(x_, wc0b_ref[...], preferred_element_type=jnp.float32)
          + bc0_ref[...])
    mf = mf_ref[...]
    o_ref[...] = (mf * (Z * c1 + (1 - Z) * c0)
                  + (1 - mf) * (Z * c0 + (1 - Z) * c1))


def _dense_c(o2, x_, mf, b_gat, gn_gamma, gn_beta, gn_alpha,
             W_c0, b_c0, W_c1, b_c1):
    return pl.pallas_call(
        _dense_c_body,
        out_shape=jax.ShapeDtypeStruct((N, D), jnp.float32),
    )(o2, x_, mf, b_gat.reshape(1, D), gn_gamma.reshape(1, D),
      gn_beta.reshape(1, D), gn_alpha.reshape(1, D),
      W_c0[:D], W_c0[D:], b_c0.reshape(1, D),
      W_c1[:D], W_c1[D:], b_c1.reshape(1, D))


def kernel(x_, edge_index, edge_attr, mask, W_t0, b_t0, W_t1, b_t1, W_gat,
           att_src, att_dst, b_gat, gn_gamma, gn_beta, gn_alpha,
           W_c0, b_c0, W_c1, b_c1):
    mf = mask.astype(jnp.float32).reshape(N, 1)

    h2, a_src, a_dst = _dense_a(x_, mf, W_t0, b_t0, W_t1, b_t1, W_gat,
                                att_src, att_dst)

    src = edge_index[0].astype(jnp.int32)
    dst = edge_index[1].astype(jnp.int32)
    pad = EP - E
    srcp = jnp.pad(src, (0, pad)).reshape(EP // C, C)
    dstp = jnp.pad(dst, (0, pad)).reshape(EP // C, C)
    attrp = jnp.pad(edge_attr.astype(jnp.float32), (0, pad)).reshape(
        EP // C, C)

    outp = _sc_edge(h2, a_src.reshape(N), a_dst.reshape(N),
                    srcp, dstp, attrp)

    o2 = outp.reshape(NC, NPAD, H)[:, :N, :]
    return _dense_c(o2, x_, mf, b_gat, gn_gamma, gn_beta, gn_alpha,
                    W_c0, b_c0, W_c1, b_c1)
